# Initial kernel scaffold; baseline (speedup 1.0000x reference)
import jax
import jax.numpy as jnp
from jax.experimental import pallas as pl


def kernel(input_2d_tensor, substitution_tensor):
    raise NotImplementedError("write your pallas kernel here")


# speedup vs baseline: 59.2835x; 59.2835x over previous
import functools

import jax
import jax.numpy as jnp
from jax.experimental import pallas as pl

R = 262144
C = 128
BLOCK_R = 8192


def _broadcast_body(sub_ref, out_ref):
    out_ref[...] = jnp.broadcast_to(sub_ref[...], out_ref.shape)


def kernel(input_2d_tensor, substitution_tensor):
    num_rows, num_cols = input_2d_tensor.shape
    grid = (num_rows // BLOCK_R,)
    return pl.pallas_call(
        _broadcast_body,
        grid=grid,
        in_specs=[pl.BlockSpec((num_cols,), lambda i: (0,))],
        out_specs=pl.BlockSpec((BLOCK_R, num_cols), lambda i: (i, 0)),
        out_shape=jax.ShapeDtypeStruct((num_rows, num_cols),
                                       substitution_tensor.dtype),
    )(substitution_tensor)
